# initial kernel scaffold (unmeasured)
import jax
import jax.numpy as jnp
from jax import lax
from jax.experimental import pallas as pl
from jax.experimental.pallas import tpu as pltpu

N_DEV = 32
N_TOK = 512
D_IN = 256
D_OUT = 512
E_LOC = 4
CAP = 3
SLOTS = E_LOC * CAP
TOT_SLOTS = N_DEV * SLOTS
ROWS_OUT = N_TOK // N_DEV


def kernel(x, router_W, route_idx, expert_W):
    del router_W
    e_col = route_idx.astype(jnp.int32)
    e_row = e_col.reshape(1, N_TOK)

    def body(x_ref, e_col_ref, e_row_ref, ew_ref, out_ref,
             gather_buf, send_sems, recv_sems):
        me = lax.axis_index("i")

        e_c = e_col_ref[:, :]
        e_r = e_row_ref[:, :]
        rowi = lax.broadcasted_iota(jnp.int32, (N_TOK, N_TOK), 0)
        coli = lax.broadcasted_iota(jnp.int32, (N_TOK, N_TOK), 1)
        eqm = e_c == e_r
        rank_r = jnp.sum((eqm & (rowi < coli)).astype(jnp.float32),
                         axis=0, keepdims=True)
        rank_c = jnp.sum((eqm & (coli < rowi)).astype(jnp.float32),
                         axis=1, keepdims=True)

        s_row = lax.broadcasted_iota(jnp.int32, (SLOTS, N_TOK), 0)
        k_of_s = s_row // CAP
        r_of_s = s_row % CAP
        sel = (e_r == me * E_LOC + k_of_s) & \
              (rank_r.astype(jnp.int32) == r_of_s)
        P = sel.astype(jnp.bfloat16)
        xb = x_ref[:, :].astype(jnp.bfloat16)
        cx = jnp.dot(P, xb, preferred_element_type=jnp.float32)
        k_col = lax.broadcasted_iota(jnp.int32, (SLOTS, 1), 0) // CAP
        block = jnp.zeros((SLOTS, D_OUT), jnp.float32)
        for k in range(E_LOC):
            ck = (cx * (k_col == k).astype(jnp.float32)).astype(jnp.bfloat16)
            wk = ew_ref[k].astype(jnp.bfloat16)
            block = block + jnp.dot(ck, wk,
                                    preferred_element_type=jnp.float32)
        gather_buf[pl.ds(me * SLOTS, SLOTS), :] = block.astype(jnp.bfloat16)

        sends = []
        for o in range(1, N_DEV):
            dst = lax.rem(me + o, N_DEV)
            r = pltpu.make_async_remote_copy(
                src_ref=gather_buf.at[pl.ds(me * SLOTS, SLOTS), :],
                dst_ref=gather_buf.at[pl.ds(me * SLOTS, SLOTS), :],
                send_sem=send_sems.at[o],
                recv_sem=recv_sems.at[o],
                device_id=(dst,),
                device_id_type=pl.DeviceIdType.MESH,
            )
            r.start()
            sends.append(r)

        s_col = lax.broadcasted_iota(jnp.int32, (N_TOK, TOT_SLOTS), 1)
        slot_c = CAP * e_c + rank_c.astype(jnp.int32)
        Q = ((s_col == slot_c) & (rank_c < float(CAP))).astype(jnp.bfloat16)
        j_row = lax.broadcasted_iota(jnp.int32, (ROWS_OUT, N_TOK), 0)
        t_col = lax.broadcasted_iota(jnp.int32, (ROWS_OUT, N_TOK), 1)
        Tsel = (t_col == me * ROWS_OUT + j_row).astype(jnp.bfloat16)
        P_out = jnp.dot(Tsel, Q, preferred_element_type=jnp.float32)

        for o in range(1, N_DEV):
            src = lax.rem(me - o + N_DEV, N_DEV)
            rr = pltpu.make_async_remote_copy(
                src_ref=gather_buf.at[pl.ds(src * SLOTS, SLOTS), :],
                dst_ref=gather_buf.at[pl.ds(src * SLOTS, SLOTS), :],
                send_sem=send_sems.at[o],
                recv_sem=recv_sems.at[o],
                device_id=(me,),
                device_id_type=pl.DeviceIdType.MESH,
            )
            rr.wait_recv()

        out_ref[:, :] = jnp.dot(P_out.astype(jnp.bfloat16),
                                gather_buf[:, :],
                                preferred_element_type=jnp.float32)

        for r in sends:
            r.wait_send()

    return pl.pallas_call(
        body,
        out_shape=jax.ShapeDtypeStruct((ROWS_OUT, D_OUT), jnp.float32),
        in_specs=[
            pl.BlockSpec(memory_space=pltpu.VMEM),
            pl.BlockSpec(memory_space=pltpu.VMEM),
            pl.BlockSpec(memory_space=pltpu.VMEM),
            pl.BlockSpec(memory_space=pltpu.VMEM),
        ],
        out_specs=pl.BlockSpec(memory_space=pltpu.VMEM),
        scratch_shapes=[
            pltpu.VMEM((TOT_SLOTS, D_OUT), jnp.bfloat16),
            pltpu.SemaphoreType.DMA((N_DEV,)),
            pltpu.SemaphoreType.DMA((N_DEV,)),
        ],
        compiler_params=pltpu.CompilerParams(collective_id=0),
    )(x, e_col, e_row, expert_W)


# baseline (device time: 25904 ns/iter reference)
import jax
import jax.numpy as jnp
from jax import lax
from jax.experimental import pallas as pl
from jax.experimental.pallas import tpu as pltpu

N_DEV = 32
N_TOK = 512
D_IN = 256
D_OUT = 512
E_LOC = 4
CAP = 3
SLOTS = E_LOC * CAP
SLOTS_PAD = 16
TOT_SLOTS = N_DEV * SLOTS_PAD
ROWS_OUT = N_TOK // N_DEV


def kernel(x, router_W, route_idx, expert_W):
    del router_W
    e_col = route_idx.astype(jnp.int32)
    e_row = e_col.reshape(1, N_TOK)

    def body(x_ref, e_col_ref, e_row_ref, ew_ref, out_ref,
             gather_buf, send_sems, recv_sems):
        me = lax.axis_index("i")

        e_c = e_col_ref[:, :]
        e_r = e_row_ref[:, :]
        rowi = lax.broadcasted_iota(jnp.int32, (N_TOK, N_TOK), 0)
        coli = lax.broadcasted_iota(jnp.int32, (N_TOK, N_TOK), 1)
        eqm = e_c == e_r
        rank_r = jnp.sum((eqm & (rowi < coli)).astype(jnp.float32),
                         axis=0, keepdims=True)
        rank_c = jnp.sum((eqm & (coli < rowi)).astype(jnp.float32),
                         axis=1, keepdims=True)

        s_row = lax.broadcasted_iota(jnp.int32, (SLOTS_PAD, N_TOK), 0)
        k_of_s = s_row // CAP
        r_of_s = s_row % CAP
        sel = (e_r == me * E_LOC + k_of_s) & \
              (rank_r.astype(jnp.int32) == r_of_s) & (s_row < SLOTS)
        P = sel.astype(jnp.bfloat16)
        xb = x_ref[:, :].astype(jnp.bfloat16)
        cx = jnp.dot(P, xb, preferred_element_type=jnp.float32)
        k_col = lax.broadcasted_iota(jnp.int32, (SLOTS_PAD, 1), 0) // CAP
        block = jnp.zeros((SLOTS_PAD, D_OUT), jnp.float32)
        for k in range(E_LOC):
            ck = (cx * (k_col == k).astype(jnp.float32)).astype(jnp.bfloat16)
            wk = ew_ref[k].astype(jnp.bfloat16)
            block = block + jnp.dot(ck, wk,
                                    preferred_element_type=jnp.float32)
        gather_buf[pl.ds(me * SLOTS_PAD, SLOTS_PAD), :] = \
            block.astype(jnp.bfloat16)

        sends = []
        for o in range(1, N_DEV):
            dst = lax.rem(me + o, N_DEV)
            r = pltpu.make_async_remote_copy(
                src_ref=gather_buf.at[pl.ds(me * SLOTS_PAD, SLOTS_PAD), :],
                dst_ref=gather_buf.at[pl.ds(me * SLOTS_PAD, SLOTS_PAD), :],
                send_sem=send_sems.at[o],
                recv_sem=recv_sems.at[o],
                device_id=(dst,),
                device_id_type=pl.DeviceIdType.MESH,
            )
            r.start()
            sends.append(r)

        s_col = lax.broadcasted_iota(jnp.int32, (N_TOK, TOT_SLOTS), 1)
        slot_c = (SLOTS_PAD * (e_c // E_LOC) + CAP * (e_c % E_LOC)
                  + rank_c.astype(jnp.int32))
        Q = ((s_col == slot_c) & (rank_c < float(CAP))).astype(jnp.bfloat16)
        j_row = lax.broadcasted_iota(jnp.int32, (ROWS_OUT, N_TOK), 0)
        t_col = lax.broadcasted_iota(jnp.int32, (ROWS_OUT, N_TOK), 1)
        Tsel = (t_col == me * ROWS_OUT + j_row).astype(jnp.bfloat16)
        P_out = jnp.dot(Tsel, Q, preferred_element_type=jnp.float32)

        for o in range(1, N_DEV):
            src = lax.rem(me - o + N_DEV, N_DEV)
            rr = pltpu.make_async_remote_copy(
                src_ref=gather_buf.at[pl.ds(src * SLOTS_PAD, SLOTS_PAD), :],
                dst_ref=gather_buf.at[pl.ds(src * SLOTS_PAD, SLOTS_PAD), :],
                send_sem=send_sems.at[o],
                recv_sem=recv_sems.at[o],
                device_id=(me,),
                device_id_type=pl.DeviceIdType.MESH,
            )
            rr.wait_recv()

        out_ref[:, :] = jnp.dot(P_out.astype(jnp.bfloat16),
                                gather_buf[:, :],
                                preferred_element_type=jnp.float32)

        for r in sends:
            r.wait_send()

    return pl.pallas_call(
        body,
        out_shape=jax.ShapeDtypeStruct((ROWS_OUT, D_OUT), jnp.float32),
        in_specs=[
            pl.BlockSpec(memory_space=pltpu.VMEM),
            pl.BlockSpec(memory_space=pltpu.VMEM),
            pl.BlockSpec(memory_space=pltpu.VMEM),
            pl.BlockSpec(memory_space=pltpu.VMEM),
        ],
        out_specs=pl.BlockSpec(memory_space=pltpu.VMEM),
        scratch_shapes=[
            pltpu.VMEM((TOT_SLOTS, D_OUT), jnp.bfloat16),
            pltpu.SemaphoreType.DMA((N_DEV,)),
            pltpu.SemaphoreType.DMA((N_DEV,)),
        ],
    )(x, e_col, e_row, expert_W)


# device time: 20977 ns/iter; 1.2349x vs baseline; 1.2349x over previous
import jax
import jax.numpy as jnp
from jax import lax
from jax.experimental import pallas as pl
from jax.experimental.pallas import tpu as pltpu

N_DEV = 32
N_TOK = 512
D_IN = 256
D_OUT = 512
E_LOC = 4
CAP = 3
SLOTS = E_LOC * CAP
SLOTS_PAD = 16
TOT_SLOTS = N_DEV * SLOTS_PAD
ROWS_OUT = N_TOK // N_DEV


def kernel(x, router_W, route_idx, expert_W):
    del router_W
    e_col = route_idx.astype(jnp.int32)
    e_row = e_col.reshape(1, N_TOK)

    def body(x_ref, e_col_ref, e_row_ref, ew_ref, out_ref,
             gather_buf, send_sems, recv_sems):
        me = lax.axis_index("i")

        barrier_sem = pltpu.get_barrier_semaphore()
        for o in range(1, N_DEV):
            dst = lax.rem(me + o, N_DEV)
            pl.semaphore_signal(
                barrier_sem, inc=1,
                device_id=(dst,), device_id_type=pl.DeviceIdType.MESH,
            )

        e_c = e_col_ref[:, :]
        e_r = e_row_ref[:, :]
        rowi = lax.broadcasted_iota(jnp.int32, (N_TOK, N_TOK), 0)
        coli = lax.broadcasted_iota(jnp.int32, (N_TOK, N_TOK), 1)
        eqm = e_c == e_r
        rank_r = jnp.sum((eqm & (rowi < coli)).astype(jnp.float32),
                         axis=0, keepdims=True)
        rank_c = jnp.sum((eqm & (coli < rowi)).astype(jnp.float32),
                         axis=1, keepdims=True)

        s_row = lax.broadcasted_iota(jnp.int32, (SLOTS_PAD, N_TOK), 0)
        k_of_s = s_row // CAP
        r_of_s = s_row % CAP
        sel = (e_r == me * E_LOC + k_of_s) & \
              (rank_r.astype(jnp.int32) == r_of_s) & (s_row < SLOTS)
        P = sel.astype(jnp.bfloat16)
        xb = x_ref[:, :].astype(jnp.bfloat16)
        cx = jnp.dot(P, xb, preferred_element_type=jnp.float32)
        k_col = lax.broadcasted_iota(jnp.int32, (SLOTS_PAD, 1), 0) // CAP
        block = jnp.zeros((SLOTS_PAD, D_OUT), jnp.float32)
        for k in range(E_LOC):
            ck = (cx * (k_col == k).astype(jnp.float32)).astype(jnp.bfloat16)
            wk = ew_ref[k].astype(jnp.bfloat16)
            block = block + jnp.dot(ck, wk,
                                    preferred_element_type=jnp.float32)
        gather_buf[pl.ds(me * SLOTS_PAD, SLOTS_PAD), :] = \
            block.astype(jnp.bfloat16)

        pl.semaphore_wait(barrier_sem, N_DEV - 1)
        sends = []
        for o in range(1, N_DEV):
            dst = lax.rem(me + o, N_DEV)
            r = pltpu.make_async_remote_copy(
                src_ref=gather_buf.at[pl.ds(me * SLOTS_PAD, SLOTS_PAD), :],
                dst_ref=gather_buf.at[pl.ds(me * SLOTS_PAD, SLOTS_PAD), :],
                send_sem=send_sems.at[o],
                recv_sem=recv_sems.at[o],
                device_id=(dst,),
                device_id_type=pl.DeviceIdType.MESH,
            )
            r.start()
            sends.append(r)

        s_col = lax.broadcasted_iota(jnp.int32, (N_TOK, TOT_SLOTS), 1)
        slot_c = (SLOTS_PAD * (e_c // E_LOC) + CAP * (e_c % E_LOC)
                  + rank_c.astype(jnp.int32))
        Q = ((s_col == slot_c) & (rank_c < float(CAP))).astype(jnp.bfloat16)
        j_row = lax.broadcasted_iota(jnp.int32, (ROWS_OUT, N_TOK), 0)
        t_col = lax.broadcasted_iota(jnp.int32, (ROWS_OUT, N_TOK), 1)
        Tsel = (t_col == me * ROWS_OUT + j_row).astype(jnp.bfloat16)
        P_out = jnp.dot(Tsel, Q, preferred_element_type=jnp.float32)

        for o in range(1, N_DEV):
            src = lax.rem(me - o + N_DEV, N_DEV)
            rr = pltpu.make_async_remote_copy(
                src_ref=gather_buf.at[pl.ds(src * SLOTS_PAD, SLOTS_PAD), :],
                dst_ref=gather_buf.at[pl.ds(src * SLOTS_PAD, SLOTS_PAD), :],
                send_sem=send_sems.at[o],
                recv_sem=recv_sems.at[o],
                device_id=(me,),
                device_id_type=pl.DeviceIdType.MESH,
            )
            rr.wait_recv()

        out_ref[:, :] = jnp.dot(P_out.astype(jnp.bfloat16),
                                gather_buf[:, :],
                                preferred_element_type=jnp.float32)

        for r in sends:
            r.wait_send()

    return pl.pallas_call(
        body,
        out_shape=jax.ShapeDtypeStruct((ROWS_OUT, D_OUT), jnp.float32),
        in_specs=[
            pl.BlockSpec(memory_space=pltpu.VMEM),
            pl.BlockSpec(memory_space=pltpu.VMEM),
            pl.BlockSpec(memory_space=pltpu.VMEM),
            pl.BlockSpec(memory_space=pltpu.VMEM),
        ],
        out_specs=pl.BlockSpec(memory_space=pltpu.VMEM),
        scratch_shapes=[
            pltpu.VMEM((TOT_SLOTS, D_OUT), jnp.bfloat16),
            pltpu.SemaphoreType.DMA((N_DEV,)),
            pltpu.SemaphoreType.DMA((N_DEV,)),
        ],
        compiler_params=pltpu.CompilerParams(collective_id=0),
    )(x, e_col, e_row, expert_W)


# device time: 20647 ns/iter; 1.2546x vs baseline; 1.0160x over previous
import jax
import jax.numpy as jnp
from jax import lax
from jax.experimental import pallas as pl
from jax.experimental.pallas import tpu as pltpu

N_DEV = 32
N_TOK = 512
D_IN = 256
D_OUT = 512
E_LOC = 4
CAP = 3
SLOTS = E_LOC * CAP
SLOTS_PAD = 16
TOT_SLOTS = N_DEV * SLOTS_PAD
ROWS_OUT = N_TOK // N_DEV


def kernel(x, router_W, route_idx, expert_W):
    del router_W
    e_col = route_idx.astype(jnp.int32)
    e_row = e_col.reshape(1, N_TOK)

    def body(x_ref, e_col_ref, e_row_ref, ew_ref, out_ref,
             gather_buf, send_sems, recv_sems):
        me = lax.axis_index("i")

        barrier_sem = pltpu.get_barrier_semaphore()
        for o in range(1, N_DEV):
            dst = lax.rem(me + o, N_DEV)
            pl.semaphore_signal(
                barrier_sem, inc=1,
                device_id=(dst,), device_id_type=pl.DeviceIdType.MESH,
            )

        e_c = e_col_ref[:, :]
        e_r = e_row_ref[:, :]
        rowi = lax.broadcasted_iota(jnp.int32, (N_TOK, N_TOK), 0)
        coli = lax.broadcasted_iota(jnp.int32, (N_TOK, N_TOK), 1)
        eqm = e_c == e_r
        rank_r = jnp.sum((eqm & (rowi < coli)).astype(jnp.float32),
                         axis=0, keepdims=True)
        rank_c = jnp.sum((eqm & (coli < rowi)).astype(jnp.float32),
                         axis=1, keepdims=True)

        s_row = lax.broadcasted_iota(jnp.int32, (SLOTS_PAD, N_TOK), 0)
        k_of_s = s_row // CAP
        r_of_s = s_row % CAP
        sel = (e_r == me * E_LOC + k_of_s) & \
              (rank_r.astype(jnp.int32) == r_of_s) & (s_row < SLOTS)
        P = sel.astype(jnp.bfloat16)
        xb = x_ref[:, :].astype(jnp.bfloat16)
        cx = jnp.dot(P, xb, preferred_element_type=jnp.float32)
        k_col = lax.broadcasted_iota(jnp.int32, (SLOTS_PAD, 1), 0) // CAP
        block = jnp.zeros((SLOTS_PAD, D_OUT), jnp.float32)
        for k in range(E_LOC):
            ck = (cx * (k_col == k).astype(jnp.float32)).astype(jnp.bfloat16)
            wk = ew_ref[k].astype(jnp.bfloat16)
            block = block + jnp.dot(ck, wk,
                                    preferred_element_type=jnp.float32)
        gather_buf[pl.ds(me * SLOTS_PAD, SLOTS_PAD), :] = \
            block.astype(jnp.bfloat16)

        s_col = lax.broadcasted_iota(jnp.int32, (N_TOK, TOT_SLOTS), 1)
        slot_c = (SLOTS_PAD * (e_c // E_LOC) + CAP * (e_c % E_LOC)
                  + rank_c.astype(jnp.int32))
        Q = ((s_col == slot_c) & (rank_c < float(CAP))).astype(jnp.bfloat16)
        j_row = lax.broadcasted_iota(jnp.int32, (ROWS_OUT, N_TOK), 0)
        t_col = lax.broadcasted_iota(jnp.int32, (ROWS_OUT, N_TOK), 1)
        Tsel = (t_col == me * ROWS_OUT + j_row).astype(jnp.bfloat16)
        P_out = jnp.dot(Tsel, Q, preferred_element_type=jnp.float32)

        pl.semaphore_wait(barrier_sem, N_DEV - 1)
        sends = []
        for o in range(1, N_DEV):
            dst = lax.rem(me + o, N_DEV)
            r = pltpu.make_async_remote_copy(
                src_ref=gather_buf.at[pl.ds(me * SLOTS_PAD, SLOTS_PAD), :],
                dst_ref=gather_buf.at[pl.ds(me * SLOTS_PAD, SLOTS_PAD), :],
                send_sem=send_sems.at[o],
                recv_sem=recv_sems.at[o],
                device_id=(dst,),
                device_id_type=pl.DeviceIdType.MESH,
            )
            r.start()
            sends.append(r)

        for o in range(1, N_DEV):
            src = lax.rem(me - o + N_DEV, N_DEV)
            rr = pltpu.make_async_remote_copy(
                src_ref=gather_buf.at[pl.ds(src * SLOTS_PAD, SLOTS_PAD), :],
                dst_ref=gather_buf.at[pl.ds(src * SLOTS_PAD, SLOTS_PAD), :],
                send_sem=send_sems.at[o],
                recv_sem=recv_sems.at[o],
                device_id=(me,),
                device_id_type=pl.DeviceIdType.MESH,
            )
            rr.wait_recv()

        out_ref[:, :] = jnp.dot(P_out.astype(jnp.bfloat16),
                                gather_buf[:, :],
                                preferred_element_type=jnp.float32)

        for r in sends:
            r.wait_send()

    return pl.pallas_call(
        body,
        out_shape=jax.ShapeDtypeStruct((ROWS_OUT, D_OUT), jnp.float32),
        in_specs=[
            pl.BlockSpec(memory_space=pltpu.VMEM),
            pl.BlockSpec(memory_space=pltpu.VMEM),
            pl.BlockSpec(memory_space=pltpu.VMEM),
            pl.BlockSpec(memory_space=pltpu.VMEM),
        ],
        out_specs=pl.BlockSpec(memory_space=pltpu.VMEM),
        scratch_shapes=[
            pltpu.VMEM((TOT_SLOTS, D_OUT), jnp.bfloat16),
            pltpu.SemaphoreType.DMA((N_DEV,)),
            pltpu.SemaphoreType.DMA((N_DEV,)),
        ],
        compiler_params=pltpu.CompilerParams(collective_id=0),
    )(x, e_col, e_row, expert_W)
